# baseline (speedup 1.0000x reference)
import functools

import jax
import jax.numpy as jnp
from jax import lax
from jax.experimental import pallas as pl
from jax.experimental.pallas import tpu as pltpu
from jax.experimental.pallas import tpu_sc as plsc

_VOCAB = 1_000_000
_DIM = 64
_PAD = 128
_B = 4096
_L = 200

_NC = 2
_NS = 16
_NW = _NC * _NS
_RPW = _B // _NW
_NB = 4
_DW = 1
_DG = _NB - _DW


_NCOLS = 7812
_CPW = _NCOLS // _NW


def _untile_body(tt_hbm, tail_hbm, out_hbm, in_v, tr0_v, tr1_v, base_v, gsem, wsem):
    wid = lax.axis_index("s") * _NC + lax.axis_index("c")

    for k in range(8):
        lane = lax.iota(jnp.int32, 16)
        base_v[k * 16:(k + 1) * 16] = (lane + k * 16) * _DIM

    def col(i):
        return wid + i * _NW

    def start_in(i, buf):
        pltpu.async_copy(
            tt_hbm.at[:, pl.ds(col(i) * 128, 128)], in_v.at[buf], gsem)

    def wait_in(i, buf):
        pltpu.make_async_copy(
            tt_hbm.at[:, pl.ds(col(i) * 128, 128)], in_v.at[buf],
            gsem).wait()

    trs = (tr0_v, tr1_v)

    def transpose(buf):
        for k in range(8):
            base = base_v[pl.ds(k * 16, 16)]
            for d in range(_DIM):
                v = in_v[buf, d, pl.ds(k * 16, 16)]
                plsc.store_scatter(trs[buf], [base + d], v)

    def start_wb(i, buf):
        pltpu.async_copy(
            trs[buf], out_hbm.at[pl.ds(col(i) * 8192, 8192)], wsem)

    def wait_wb(i, buf):
        pltpu.make_async_copy(
            trs[buf], out_hbm.at[pl.ds(col(i) * 8192, 8192)],
            wsem).wait()

    start_in(0, 0)
    start_in(1, 1)
    wait_in(0, 0)
    transpose(0)
    start_wb(0, 0)

    @pl.loop(1, _CPW - 1, step=2)
    def _steady(io):
        for b in range(2):
            i = io + b
            buf = (1 + b) % 2
            start_in(i + 1, (buf + 1) % 2)
            wait_in(i, buf)
            wait_wb(i - 1, (buf + 1) % 2)
            transpose(buf)
            start_wb(i, buf)

    last = _CPW - 1
    wait_in(last, last % 2)
    wait_wb(last - 1, (last - 1) % 2)
    transpose(last % 2)
    start_wb(last, last % 2)
    wait_wb(last, last % 2)

    @pl.when(wid < 4)
    def _extra():
        c = _CPW * _NW + wid
        pltpu.sync_copy(tt_hbm.at[:, pl.ds(c * 128, 128)], in_v.at[0])
        transpose(0)
        pltpu.sync_copy(tr0_v, out_hbm.at[pl.ds(c * 8192, 8192)])

    @pl.when(wid == 4)
    def _partial():
        pltpu.sync_copy(tail_hbm, in_v.at[0])
        for k in range(4):
            base = base_v[pl.ds(k * 16, 16)]
            for d in range(_DIM):
                v = in_v[0, d, pl.ds(k * 16, 16)]
                plsc.store_scatter(tr0_v, [base + d], v)
        pltpu.sync_copy(
            tr0_v.at[pl.ds(0, 4096)],
            out_hbm.at[pl.ds(_NCOLS * 8192, 4096)])


@jax.jit
def _table_linearize(tt, tail):
    mesh = plsc.VectorSubcoreMesh(core_axis_name="c", subcore_axis_name="s")
    fn = functools.partial(
        pl.kernel,
        out_type=jax.ShapeDtypeStruct((_VOCAB * _DIM,), jnp.float32),
        mesh=mesh,
        scratch_types=[
            pltpu.VMEM((2, _DIM, 128), jnp.float32),
            pltpu.VMEM((8192,), jnp.float32),
            pltpu.VMEM((8192,), jnp.float32),
            pltpu.VMEM((128,), jnp.int32),
            pltpu.SemaphoreType.DMA,
            pltpu.SemaphoreType.DMA,
        ],
        compiler_params=pltpu.CompilerParams(
            use_tc_tiling_on_sc=True, needs_layout_passes=False),
    )(_untile_body)
    return fn(tt, tail)


def _emb_body(idx_hbm, table_hbm, out_hbm, idx_v, rows_v, gsem, wsem):
    wid = lax.axis_index("s") * _NC + lax.axis_index("c")
    base = wid * _RPW

    pltpu.sync_copy(idx_hbm.at[pl.ds(base, _RPW)], idx_v)

    def start_gather(g, slot):
        pltpu.async_copy(table_hbm.at[idx_v.at[g]], rows_v.at[slot], gsem)

    def wait_gather(g, slot):
        pltpu.make_async_copy(
            table_hbm.at[idx_v.at[g]], rows_v.at[slot], gsem).wait()

    def start_wb(g, slot):
        pltpu.async_copy(
            rows_v.at[slot], out_hbm.at[base + g, :, pl.ds(0, _DIM)], wsem)

    def wait_wb(g, slot):
        pltpu.make_async_copy(
            rows_v.at[slot], out_hbm.at[base + g, :, pl.ds(0, _DIM)],
            wsem).wait()

    for g in range(_DG):
        start_gather(g, g)

    def step(g, b):
        wait_gather(g, b)
        start_wb(g, b)
        wait_wb(g - _DW, (b - _DW) % _NB)
        start_gather(g + _DG, (b + _DG) % _NB)

    for g in range(_NB):
        wait_gather(g, g)
        start_wb(g, g)
        if g >= _DW:
            wait_wb(g - _DW, (g - _DW) % _NB)
        start_gather(g + _DG, (g + _DG) % _NB)

    @pl.loop(_NB, _RPW - _NB, step=_NB)
    def _steady(go):
        for b in range(_NB):
            step(go + b, b)

    for g in range(_RPW - _NB, _RPW):
        wait_gather(g, g % _NB)
        start_wb(g, g % _NB)
        wait_wb(g - _DW, (g - _DW) % _NB)
        if g + _DG < _RPW:
            start_gather(g + _DG, (g + _DG) % _NB)

    for g in range(_RPW - _DW, _RPW):
        wait_wb(g, g % _NB)


@jax.jit
def _embedding_lookup(idx, table):
    mesh = plsc.VectorSubcoreMesh(core_axis_name="c", subcore_axis_name="s")
    fn = functools.partial(
        pl.kernel,
        out_type=jax.ShapeDtypeStruct((_B, _L, _PAD), jnp.float32),
        mesh=mesh,
        scratch_types=[
            pltpu.VMEM((_RPW, _L), jnp.int32),
            pltpu.VMEM((_NB, _L, _DIM), jnp.float32),
            pltpu.SemaphoreType.DMA,
            pltpu.SemaphoreType.DMA,
        ],
        compiler_params=pltpu.CompilerParams(use_tc_tiling_on_sc=False),
    )(_emb_body)
    return fn(idx, table)


def kernel(inputs, table):
    tail = jnp.pad(table.T[:, _NCOLS * 128:], ((0, 0), (0, 64)))
    tlin = jnp.reshape(_table_linearize(table.T, tail), (_VOCAB, _DIM))
    out128 = _embedding_lookup(inputs.astype(jnp.int32), tlin)
    return lax.slice(out128, (0, 0, 0), (_B, _L, _DIM))


# speedup vs baseline: 1.7695x; 1.7695x over previous
import functools

import jax
import jax.numpy as jnp
from jax import lax
from jax.experimental import pallas as pl
from jax.experimental.pallas import tpu as pltpu
from jax.experimental.pallas import tpu_sc as plsc

_VOCAB = 1_000_000
_DIM = 64
_PAD = 128
_B = 4096
_L = 200

_NC = 2
_NS = 16
_NW = _NC * _NS
_RPW = _B // _NW
_NB = 4
_DW = 1
_DG = _NB - _DW


def _emb_body(idx_hbm, table_hbm, out_hbm, idx_v, rows_v, gsem, wsem):
    wid = lax.axis_index("s") * _NC + lax.axis_index("c")
    base = wid * _RPW

    pltpu.sync_copy(idx_hbm.at[pl.ds(base, _RPW)], idx_v)

    def start_gather(g, slot):
        pltpu.async_copy(table_hbm.at[idx_v.at[g]], rows_v.at[slot], gsem)

    def wait_gather(g, slot):
        pltpu.make_async_copy(
            table_hbm.at[idx_v.at[g]], rows_v.at[slot], gsem).wait()

    def start_wb(g, slot):
        pltpu.async_copy(
            rows_v.at[slot], out_hbm.at[base + g, :, pl.ds(0, _DIM)], wsem)

    def wait_wb(g, slot):
        pltpu.make_async_copy(
            rows_v.at[slot], out_hbm.at[base + g, :, pl.ds(0, _DIM)],
            wsem).wait()

    for g in range(_DG):
        start_gather(g, g)

    def step(g, b):
        wait_gather(g, b)
        start_wb(g, b)
        wait_wb(g - _DW, (b - _DW) % _NB)
        start_gather(g + _DG, (b + _DG) % _NB)

    for g in range(_NB):
        wait_gather(g, g)
        start_wb(g, g)
        if g >= _DW:
            wait_wb(g - _DW, (g - _DW) % _NB)
        start_gather(g + _DG, (g + _DG) % _NB)

    @pl.loop(_NB, _RPW - _NB, step=_NB)
    def _steady(go):
        for b in range(_NB):
            step(go + b, b)

    for g in range(_RPW - _NB, _RPW):
        wait_gather(g, g % _NB)
        start_wb(g, g % _NB)
        wait_wb(g - _DW, (g - _DW) % _NB)
        if g + _DG < _RPW:
            start_gather(g + _DG, (g + _DG) % _NB)

    for g in range(_RPW - _DW, _RPW):
        wait_wb(g, g % _NB)


@jax.jit
def _embedding_lookup(idx, table):
    mesh = plsc.VectorSubcoreMesh(core_axis_name="c", subcore_axis_name="s")
    fn = functools.partial(
        pl.kernel,
        out_type=jax.ShapeDtypeStruct((_B, _L, _PAD), jnp.float32),
        mesh=mesh,
        scratch_types=[
            pltpu.VMEM((_RPW, _L), jnp.int32),
            pltpu.VMEM((_NB, _L, _DIM), jnp.float32),
            pltpu.SemaphoreType.DMA,
            pltpu.SemaphoreType.DMA,
        ],
        compiler_params=pltpu.CompilerParams(use_tc_tiling_on_sc=False),
    )(_emb_body)
    return fn(idx, table)


def kernel(inputs, table):
    out128 = _embedding_lookup(inputs.astype(jnp.int32), table)
    return lax.slice(out128, (0, 0, 0), (_B, _L, _DIM))
